# initial kernel scaffold (unmeasured)
import jax
import jax.numpy as jnp
from jax import lax
from jax.experimental import pallas as pl
from jax.experimental.pallas import tpu as pltpu


def kernel(
    x,
):
    def body(*refs):
        pass

    out_shape = jax.ShapeDtypeStruct(..., jnp.float32)
    return pl.pallas_call(body, out_shape=out_shape)(...)


# baseline (device time: 427551 ns/iter reference)
import jax
import jax.numpy as jnp
from jax import lax
from jax.experimental import pallas as pl
from jax.experimental.pallas import tpu as pltpu

M = 8192
N = 2048
NH = N // 2
TILE = 512
N_TILES = M // TILE


def kernel(x):
    def body(x_ref, out_ref, vx_ref, send_sem, recv_sem, copy_sem):
        my_x = lax.axis_index("x")
        my_y = lax.axis_index("y")
        my_z = lax.axis_index("z")
        peer = (my_x, 1 - my_y, my_z)

        barrier_sem = pltpu.get_barrier_semaphore()
        pl.semaphore_signal(
            barrier_sem, inc=1, device_id=peer,
            device_id_type=pl.DeviceIdType.MESH,
        )
        pl.semaphore_wait(barrier_sem, 1)

        peer_col0 = (1 - my_y) * NH
        rdma = pltpu.make_async_remote_copy(
            src_ref=x_ref.at[0, :, pl.ds(peer_col0, NH)],
            dst_ref=out_ref,
            send_sem=send_sem,
            recv_sem=recv_sem,
            device_id=peer,
            device_id_type=pl.DeviceIdType.MESH,
        )
        rdma.start()
        rdma.wait()

        my_col0 = my_y * NH
        for t in range(N_TILES):
            r0 = t * TILE
            cp = pltpu.make_async_copy(
                x_ref.at[0, pl.ds(r0, TILE), pl.ds(my_col0, NH)],
                vx_ref,
                copy_sem,
            )
            cp.start()
            cp.wait()
            out_ref[pl.ds(r0, TILE), :] = out_ref[pl.ds(r0, TILE), :] + vx_ref[...]

    return pl.pallas_call(
        body,
        out_shape=jax.ShapeDtypeStruct((M, NH), jnp.float32),
        in_specs=[pl.BlockSpec(memory_space=pl.ANY)],
        out_specs=pl.BlockSpec(memory_space=pltpu.MemorySpace.VMEM),
        scratch_shapes=[
            pltpu.VMEM((TILE, NH), jnp.float32),
            pltpu.SemaphoreType.DMA,
            pltpu.SemaphoreType.DMA,
            pltpu.SemaphoreType.DMA,
        ],
        compiler_params=pltpu.CompilerParams(
            collective_id=0, vmem_limit_bytes=48 * 1024 * 1024
        ),
    )(x)


# device time: 189443 ns/iter; 2.2569x vs baseline; 2.2569x over previous
import jax
import jax.numpy as jnp
from jax import lax
from jax.experimental import pallas as pl
from jax.experimental.pallas import tpu as pltpu

M = 8192
N = 2048
NH = N // 2
QM = M // 4
C = 8
R = QM // C


def kernel(x):
    def body(
        x_ref, out_ref, vx_ref,
        send_y, recv_y,
        send_b1x, recv_b1x, send_b1z, recv_b1z,
        send_fx, recv_fx, send_fz, recv_fz,
        copy_sems,
    ):
        my_x = lax.axis_index("x")
        my_y = lax.axis_index("y")
        my_z = lax.axis_index("z")
        peer = (my_x, 1 - my_y, my_z)
        xn = (1 - my_x, my_y, my_z)
        zn = (my_x, my_y, 1 - my_z)

        q = 2 * my_x + my_z
        xq = 2 * (1 - my_x) + my_z
        zq = 2 * my_x + (1 - my_z)

        my_col0 = my_y * NH
        peer_col0 = (1 - my_y) * NH

        barrier_sem = pltpu.get_barrier_semaphore()
        for nbr in (peer, xn, zn):
            pl.semaphore_signal(
                barrier_sem, inc=1, device_id=nbr,
                device_id_type=pl.DeviceIdType.MESH,
            )
        pl.semaphore_wait(barrier_sem, 3)

        a_rdmas = []
        local_cps = []
        for c in range(C):
            rows = pl.ds(q * QM + c * R, R)
            rdma = pltpu.make_async_remote_copy(
                src_ref=x_ref.at[0, rows, pl.ds(peer_col0, NH)],
                dst_ref=out_ref.at[rows, :],
                send_sem=send_y.at[c],
                recv_sem=recv_y.at[c],
                device_id=peer,
                device_id_type=pl.DeviceIdType.MESH,
            )
            rdma.start()
            a_rdmas.append(rdma)
            cp = pltpu.make_async_copy(
                x_ref.at[0, rows, pl.ds(my_col0, NH)],
                vx_ref.at[pl.ds(c * R, R), :],
                copy_sems.at[c],
            )
            cp.start()
            local_cps.append(cp)

        b1x_rdmas = []
        b1z_rdmas = []
        for c in range(C):
            a_rdmas[c].wait_recv()
            local_cps[c].wait()
            rows = pl.ds(q * QM + c * R, R)
            out_ref[rows, :] = out_ref[rows, :] + vx_ref[pl.ds(c * R, R), :]
            for tgt, ssem, rsem, lst in (
                (xn, send_b1x, recv_b1x, b1x_rdmas),
                (zn, send_b1z, recv_b1z, b1z_rdmas),
            ):
                rdma = pltpu.make_async_remote_copy(
                    src_ref=out_ref.at[rows, :],
                    dst_ref=out_ref.at[rows, :],
                    send_sem=ssem.at[c],
                    recv_sem=rsem.at[c],
                    device_id=tgt,
                    device_id_type=pl.DeviceIdType.MESH,
                )
                rdma.start()
                lst.append(rdma)

        fwd_rdmas = []
        for c in range(C):
            if c % 2 == 0:
                b1z_rdmas[c].wait_recv()
                rows = pl.ds(zq * QM + c * R, R)
                rdma = pltpu.make_async_remote_copy(
                    src_ref=out_ref.at[rows, :],
                    dst_ref=out_ref.at[rows, :],
                    send_sem=send_fx.at[c // 2],
                    recv_sem=recv_fx.at[c // 2],
                    device_id=xn,
                    device_id_type=pl.DeviceIdType.MESH,
                )
            else:
                b1x_rdmas[c].wait_recv()
                rows = pl.ds(xq * QM + c * R, R)
                rdma = pltpu.make_async_remote_copy(
                    src_ref=out_ref.at[rows, :],
                    dst_ref=out_ref.at[rows, :],
                    send_sem=send_fz.at[c // 2],
                    recv_sem=recv_fz.at[c // 2],
                    device_id=zn,
                    device_id_type=pl.DeviceIdType.MESH,
                )
            rdma.start()
            fwd_rdmas.append(rdma)

        for c in range(C):
            if c % 2 == 0:
                b1x_rdmas[c].wait_recv()
            else:
                b1z_rdmas[c].wait_recv()
        for rdma in fwd_rdmas:
            rdma.wait_recv()
        for rdma in a_rdmas + b1x_rdmas + b1z_rdmas + fwd_rdmas:
            rdma.wait_send()

    return pl.pallas_call(
        body,
        out_shape=jax.ShapeDtypeStruct((M, NH), jnp.float32),
        in_specs=[pl.BlockSpec(memory_space=pl.ANY)],
        out_specs=pl.BlockSpec(memory_space=pltpu.MemorySpace.VMEM),
        scratch_shapes=[
            pltpu.VMEM((QM, NH), jnp.float32),
            pltpu.SemaphoreType.DMA((C,)),
            pltpu.SemaphoreType.DMA((C,)),
            pltpu.SemaphoreType.DMA((C,)),
            pltpu.SemaphoreType.DMA((C,)),
            pltpu.SemaphoreType.DMA((C,)),
            pltpu.SemaphoreType.DMA((C,)),
            pltpu.SemaphoreType.DMA((C // 2,)),
            pltpu.SemaphoreType.DMA((C // 2,)),
            pltpu.SemaphoreType.DMA((C // 2,)),
            pltpu.SemaphoreType.DMA((C // 2,)),
            pltpu.SemaphoreType.DMA((C,)),
        ],
        compiler_params=pltpu.CompilerParams(
            collective_id=0, vmem_limit_bytes=56 * 1024 * 1024
        ),
    )(x)


# device time: 176897 ns/iter; 2.4169x vs baseline; 1.0709x over previous
import jax
import jax.numpy as jnp
from jax import lax
from jax.experimental import pallas as pl
from jax.experimental.pallas import tpu as pltpu

M = 8192
N = 2048
NH = N // 2
PA = 1824
PC = 6
PR = PA // PC
U0 = 4 * PA
UC = 2
UR = (M - U0) // UC
VU0 = PA


def kernel(x):
    def body(
        x_ref, out_ref, vx_ref,
        send_yp, recv_yp, send_yu, recv_yu,
        send_b1x, recv_b1x, send_b1z, recv_b1z,
        send_fx, recv_fx, send_fz, recv_fz,
        copy_sems,
    ):
        my_x = lax.axis_index("x")
        my_y = lax.axis_index("y")
        my_z = lax.axis_index("z")
        peer = (my_x, 1 - my_y, my_z)
        xn = (1 - my_x, my_y, my_z)
        zn = (my_x, my_y, 1 - my_z)

        q = 2 * my_x + my_z
        xq = 2 * (1 - my_x) + my_z
        zq = 2 * my_x + (1 - my_z)

        my_col0 = my_y * NH
        peer_col0 = (1 - my_y) * NH

        barrier_sem = pltpu.get_barrier_semaphore()
        for nbr in (peer, xn, zn):
            pl.semaphore_signal(
                barrier_sem, inc=1, device_id=nbr,
                device_id_type=pl.DeviceIdType.MESH,
            )
        pl.semaphore_wait(barrier_sem, 3)

        yp_rdmas, yu_rdmas, cps = [], [], []
        for c in range(PC):
            rows = pl.ds(q * PA + c * PR, PR)
            rdma = pltpu.make_async_remote_copy(
                src_ref=x_ref.at[0, rows, pl.ds(peer_col0, NH)],
                dst_ref=out_ref.at[rows, :],
                send_sem=send_yp.at[c], recv_sem=recv_yp.at[c],
                device_id=peer, device_id_type=pl.DeviceIdType.MESH,
            )
            rdma.start()
            yp_rdmas.append(rdma)
            cp = pltpu.make_async_copy(
                x_ref.at[0, rows, pl.ds(my_col0, NH)],
                vx_ref.at[pl.ds(c * PR, PR), :],
                copy_sems.at[c],
            )
            cp.start()
            cps.append(cp)
        for j in range(UC):
            rows = pl.ds(U0 + j * UR, UR)
            rdma = pltpu.make_async_remote_copy(
                src_ref=x_ref.at[0, rows, pl.ds(peer_col0, NH)],
                dst_ref=out_ref.at[rows, :],
                send_sem=send_yu.at[j], recv_sem=recv_yu.at[j],
                device_id=peer, device_id_type=pl.DeviceIdType.MESH,
            )
            rdma.start()
            yu_rdmas.append(rdma)
            cp = pltpu.make_async_copy(
                x_ref.at[0, rows, pl.ds(my_col0, NH)],
                vx_ref.at[pl.ds(VU0 + j * UR, UR), :],
                copy_sems.at[PC + j],
            )
            cp.start()
            cps.append(cp)

        b1x_rdmas, b1z_rdmas = [], []
        for c in range(PC):
            yp_rdmas[c].wait_recv()
            cps[c].wait()
            rows = pl.ds(q * PA + c * PR, PR)
            out_ref[rows, :] = out_ref[rows, :] + vx_ref[pl.ds(c * PR, PR), :]
            for tgt, ssem, rsem, lst in (
                (xn, send_b1x, recv_b1x, b1x_rdmas),
                (zn, send_b1z, recv_b1z, b1z_rdmas),
            ):
                rdma = pltpu.make_async_remote_copy(
                    src_ref=out_ref.at[rows, :],
                    dst_ref=out_ref.at[rows, :],
                    send_sem=ssem.at[c], recv_sem=rsem.at[c],
                    device_id=tgt, device_id_type=pl.DeviceIdType.MESH,
                )
                rdma.start()
                lst.append(rdma)

        fwd_rdmas = []
        for c in range(PC):
            if c % 2 == 0:
                b1z_rdmas[c].wait_recv()
                rows = pl.ds(zq * PA + c * PR, PR)
                rdma = pltpu.make_async_remote_copy(
                    src_ref=out_ref.at[rows, :], dst_ref=out_ref.at[rows, :],
                    send_sem=send_fx.at[c // 2], recv_sem=recv_fx.at[c // 2],
                    device_id=xn, device_id_type=pl.DeviceIdType.MESH,
                )
            else:
                b1x_rdmas[c].wait_recv()
                rows = pl.ds(xq * PA + c * PR, PR)
                rdma = pltpu.make_async_remote_copy(
                    src_ref=out_ref.at[rows, :], dst_ref=out_ref.at[rows, :],
                    send_sem=send_fz.at[c // 2], recv_sem=recv_fz.at[c // 2],
                    device_id=zn, device_id_type=pl.DeviceIdType.MESH,
                )
            rdma.start()
            fwd_rdmas.append(rdma)

        for j in range(UC):
            yu_rdmas[j].wait_recv()
            cps[PC + j].wait()
            rows = pl.ds(U0 + j * UR, UR)
            out_ref[rows, :] = (
                out_ref[rows, :] + vx_ref[pl.ds(VU0 + j * UR, UR), :]
            )

        for c in range(PC):
            if c % 2 == 0:
                b1x_rdmas[c].wait_recv()
            else:
                b1z_rdmas[c].wait_recv()
        for rdma in fwd_rdmas:
            rdma.wait_recv()
        for rdma in yp_rdmas + yu_rdmas + b1x_rdmas + b1z_rdmas + fwd_rdmas:
            rdma.wait_send()

    return pl.pallas_call(
        body,
        out_shape=jax.ShapeDtypeStruct((M, NH), jnp.float32),
        in_specs=[pl.BlockSpec(memory_space=pl.ANY)],
        out_specs=pl.BlockSpec(memory_space=pltpu.MemorySpace.VMEM),
        scratch_shapes=[
            pltpu.VMEM((PA + M - U0, NH), jnp.float32),
            pltpu.SemaphoreType.DMA((PC,)),
            pltpu.SemaphoreType.DMA((PC,)),
            pltpu.SemaphoreType.DMA((UC,)),
            pltpu.SemaphoreType.DMA((UC,)),
            pltpu.SemaphoreType.DMA((PC,)),
            pltpu.SemaphoreType.DMA((PC,)),
            pltpu.SemaphoreType.DMA((PC,)),
            pltpu.SemaphoreType.DMA((PC,)),
            pltpu.SemaphoreType.DMA((PC // 2,)),
            pltpu.SemaphoreType.DMA((PC // 2,)),
            pltpu.SemaphoreType.DMA((PC // 2,)),
            pltpu.SemaphoreType.DMA((PC // 2,)),
            pltpu.SemaphoreType.DMA((PC + UC,)),
        ],
        compiler_params=pltpu.CompilerParams(
            collective_id=0, vmem_limit_bytes=56 * 1024 * 1024
        ),
    )(x)


# device time: 170729 ns/iter; 2.5043x vs baseline; 1.0361x over previous
import jax
import jax.numpy as jnp
from jax import lax
from jax.experimental import pallas as pl
from jax.experimental.pallas import tpu as pltpu

M = 8192
N = 2048
NH = N // 2
PA = 1824
PC = 12
PR = PA // PC
U0 = 4 * PA
UC = 4
UR = (M - U0) // UC
VU0 = PA


def kernel(x):
    def body(
        x_ref, out_ref, vx_ref,
        send_yp, recv_yp, send_yu, recv_yu,
        send_b1x, recv_b1x, send_b1z, recv_b1z,
        send_fx, recv_fx, send_fz, recv_fz,
        copy_sems,
    ):
        my_x = lax.axis_index("x")
        my_y = lax.axis_index("y")
        my_z = lax.axis_index("z")
        peer = (my_x, 1 - my_y, my_z)
        xn = (1 - my_x, my_y, my_z)
        zn = (my_x, my_y, 1 - my_z)

        q = 2 * my_x + my_z
        xq = 2 * (1 - my_x) + my_z
        zq = 2 * my_x + (1 - my_z)

        my_col0 = my_y * NH
        peer_col0 = (1 - my_y) * NH

        barrier_sem = pltpu.get_barrier_semaphore()
        for nbr in (peer, xn, zn):
            pl.semaphore_signal(
                barrier_sem, inc=1, device_id=nbr,
                device_id_type=pl.DeviceIdType.MESH,
            )
        pl.semaphore_wait(barrier_sem, 3)

        yp_rdmas, yu_rdmas, cps = [], [], []
        for c in range(PC):
            rows = pl.ds(q * PA + c * PR, PR)
            rdma = pltpu.make_async_remote_copy(
                src_ref=x_ref.at[0, rows, pl.ds(peer_col0, NH)],
                dst_ref=out_ref.at[rows, :],
                send_sem=send_yp.at[c], recv_sem=recv_yp.at[c],
                device_id=peer, device_id_type=pl.DeviceIdType.MESH,
            )
            rdma.start()
            yp_rdmas.append(rdma)
            cp = pltpu.make_async_copy(
                x_ref.at[0, rows, pl.ds(my_col0, NH)],
                vx_ref.at[pl.ds(c * PR, PR), :],
                copy_sems.at[c],
            )
            cp.start()
            cps.append(cp)
        for j in range(UC):
            rows = pl.ds(U0 + j * UR, UR)
            rdma = pltpu.make_async_remote_copy(
                src_ref=x_ref.at[0, rows, pl.ds(peer_col0, NH)],
                dst_ref=out_ref.at[rows, :],
                send_sem=send_yu.at[j], recv_sem=recv_yu.at[j],
                device_id=peer, device_id_type=pl.DeviceIdType.MESH,
            )
            rdma.start()
            yu_rdmas.append(rdma)
            cp = pltpu.make_async_copy(
                x_ref.at[0, rows, pl.ds(my_col0, NH)],
                vx_ref.at[pl.ds(VU0 + j * UR, UR), :],
                copy_sems.at[PC + j],
            )
            cp.start()
            cps.append(cp)

        b1x_rdmas, b1z_rdmas = [], []
        for c in range(PC):
            yp_rdmas[c].wait_recv()
            cps[c].wait()
            rows = pl.ds(q * PA + c * PR, PR)
            out_ref[rows, :] = out_ref[rows, :] + vx_ref[pl.ds(c * PR, PR), :]
            for tgt, ssem, rsem, lst in (
                (xn, send_b1x, recv_b1x, b1x_rdmas),
                (zn, send_b1z, recv_b1z, b1z_rdmas),
            ):
                rdma = pltpu.make_async_remote_copy(
                    src_ref=out_ref.at[rows, :],
                    dst_ref=out_ref.at[rows, :],
                    send_sem=ssem.at[c], recv_sem=rsem.at[c],
                    device_id=tgt, device_id_type=pl.DeviceIdType.MESH,
                )
                rdma.start()
                lst.append(rdma)

        fwd_rdmas = []
        for c in range(PC):
            if c % 2 == 0:
                b1z_rdmas[c].wait_recv()
                rows = pl.ds(zq * PA + c * PR, PR)
                rdma = pltpu.make_async_remote_copy(
                    src_ref=out_ref.at[rows, :], dst_ref=out_ref.at[rows, :],
                    send_sem=send_fx.at[c // 2], recv_sem=recv_fx.at[c // 2],
                    device_id=xn, device_id_type=pl.DeviceIdType.MESH,
                )
            else:
                b1x_rdmas[c].wait_recv()
                rows = pl.ds(xq * PA + c * PR, PR)
                rdma = pltpu.make_async_remote_copy(
                    src_ref=out_ref.at[rows, :], dst_ref=out_ref.at[rows, :],
                    send_sem=send_fz.at[c // 2], recv_sem=recv_fz.at[c // 2],
                    device_id=zn, device_id_type=pl.DeviceIdType.MESH,
                )
            rdma.start()
            fwd_rdmas.append(rdma)

        for j in range(UC):
            yu_rdmas[j].wait_recv()
            cps[PC + j].wait()
            rows = pl.ds(U0 + j * UR, UR)
            out_ref[rows, :] = (
                out_ref[rows, :] + vx_ref[pl.ds(VU0 + j * UR, UR), :]
            )

        for c in range(PC):
            if c % 2 == 0:
                b1x_rdmas[c].wait_recv()
            else:
                b1z_rdmas[c].wait_recv()
        for rdma in fwd_rdmas:
            rdma.wait_recv()
        for rdma in yp_rdmas + yu_rdmas + b1x_rdmas + b1z_rdmas + fwd_rdmas:
            rdma.wait_send()

    return pl.pallas_call(
        body,
        out_shape=jax.ShapeDtypeStruct((M, NH), jnp.float32),
        in_specs=[pl.BlockSpec(memory_space=pl.ANY)],
        out_specs=pl.BlockSpec(memory_space=pltpu.MemorySpace.VMEM),
        scratch_shapes=[
            pltpu.VMEM((PA + M - U0, NH), jnp.float32),
            pltpu.SemaphoreType.DMA((PC,)),
            pltpu.SemaphoreType.DMA((PC,)),
            pltpu.SemaphoreType.DMA((UC,)),
            pltpu.SemaphoreType.DMA((UC,)),
            pltpu.SemaphoreType.DMA((PC,)),
            pltpu.SemaphoreType.DMA((PC,)),
            pltpu.SemaphoreType.DMA((PC,)),
            pltpu.SemaphoreType.DMA((PC,)),
            pltpu.SemaphoreType.DMA((PC // 2,)),
            pltpu.SemaphoreType.DMA((PC // 2,)),
            pltpu.SemaphoreType.DMA((PC // 2,)),
            pltpu.SemaphoreType.DMA((PC // 2,)),
            pltpu.SemaphoreType.DMA((PC + UC,)),
        ],
        compiler_params=pltpu.CompilerParams(
            collective_id=0, vmem_limit_bytes=56 * 1024 * 1024
        ),
    )(x)
